# baseline (speedup 1.0000x reference)
import functools

import jax
import jax.numpy as jnp
from jax import lax
from jax.experimental import pallas as pl
from jax.experimental.pallas import tpu as pltpu
from jax.experimental.pallas import tpu_sc as plsc

N = 10000
E = 320000
H = 128
D_EDGE = 16
LANES = 16

NC = 2
NS = 16
NW = NC * NS
EPT = E // NW
K = 80
NCHUNK = EPT // K
NP = 10240
ZR = NP // NS


def _conv_body(x_hbm, e_hbm, src_hbm, dst_hbm, out_hbm,
               srcv0, srcv1, dstv0, dstv1,
               xrows0, xrows1, erows0, erows1, acc,
               sg0, sg1, se0, se1, ss0, ss1, si0, si1):
    c = lax.axis_index("c")
    s = lax.axis_index("s")
    wid = s * NC + c
    base = wid * EPT

    srcv = (srcv0, srcv1)
    dstv = (dstv0, dstv1)
    xrows = (xrows0, xrows1)
    erows = (erows0, erows1)
    sg = (sg0, sg1)
    se = (se0, se1)
    ss = (ss0, ss1)
    si = (si0, si1)

    def zrow(r, _):
        for j in range(H // LANES):
            xrows0[r, pl.ds(j * LANES, LANES)] = jnp.zeros((LANES,), jnp.float32)
        return 0
    lax.fori_loop(0, K, zrow, 0)
    for i in range(ZR // K):
        pltpu.async_copy(xrows0, acc.at[pl.ds(s * ZR + i * K, K)], sg0)
    for i in range(ZR // K):
        pltpu.make_async_copy(xrows0, acc.at[pl.ds(s * ZR, K)], sg0).wait()
    plsc.subcore_barrier()

    def issue_idx(slot, t):
        pltpu.async_copy(src_hbm.at[pl.ds(base + t * K, K)], srcv[slot], si[slot])
        pltpu.async_copy(dst_hbm.at[pl.ds(base + t * K, K)], dstv[slot], si[slot])

    def wait_idx(slot):
        pltpu.make_async_copy(src_hbm.at[pl.ds(base, K)], srcv[slot], si[slot]).wait()
        pltpu.make_async_copy(dst_hbm.at[pl.ds(base, K)], dstv[slot], si[slot]).wait()

    def issue_ge(slot, t):
        pltpu.async_copy(x_hbm.at[srcv[slot]], xrows[slot], sg[slot])
        pltpu.async_copy(e_hbm.at[pl.ds(base + t * K, K)], erows[slot], se[slot])

    def wait_ge(slot):
        pltpu.make_async_copy(x_hbm.at[srcv[slot]], xrows[slot], sg[slot]).wait()
        pltpu.make_async_copy(e_hbm.at[pl.ds(0, K)], erows[slot], se[slot]).wait()

    def compute(slot):
        xr = xrows[slot]
        er = erows[slot]

        def ew(q, _):
            for r in (2 * q, 2 * q + 1):
                for j in range(H // LANES):
                    sl = pl.ds(j * LANES, LANES)
                    xr[r, sl] = jnp.maximum(xr[r, sl] + er[r, sl], 0.0)
            return 0
        lax.fori_loop(0, K // 2, ew, 0)

    def scatter(slot):
        pltpu.async_copy(xrows[slot], acc.at[dstv[slot]], ss[slot], add=True)

    def wait_scatter(slot):
        pltpu.make_async_copy(xrows[slot], acc.at[dstv[slot]], ss[slot]).wait()

    issue_idx(0, 0)
    wait_idx(0)
    issue_idx(1, 1)
    issue_ge(0, 0)
    wait_ge(0)
    compute(0)
    scatter(0)
    wait_idx(1)
    issue_ge(1, 1)
    pltpu.make_async_copy(xrows[0], acc.at[dstv[0]], ss[0]).wait()
    issue_idx(0, 2)

    def pair(i, _):
        for b in range(2):
            t = 1 + 2 * i + b
            slot = (1 + b) % 2
            other = 1 - slot
            wait_ge(slot)
            compute(slot)
            scatter(slot)

            @pl.when(t < NCHUNK - 1)
            def _():
                wait_idx(other)
                issue_ge(other, t + 1)

            @pl.when(t < NCHUNK - 2)
            def _():
                wait_scatter(slot)
                issue_idx(slot, t + 2)
        return 0
    lax.fori_loop(0, (NCHUNK - 1) // 2, pair, 0)

    wait_scatter(0)
    wait_scatter(1)

    plsc.subcore_barrier()
    row0 = c * NP + s * ZR
    pltpu.sync_copy(acc.at[pl.ds(s * ZR, ZR)], out_hbm.at[pl.ds(row0, ZR)])


def _sc_conv(x, e, src, dst):
    f = pl.kernel(
        _conv_body,
        out_type=jax.ShapeDtypeStruct((2 * NP, H), jnp.float32),
        mesh=plsc.VectorSubcoreMesh(core_axis_name="c", subcore_axis_name="s"),
        scratch_types=[
            pltpu.VMEM((K,), jnp.int32),
            pltpu.VMEM((K,), jnp.int32),
            pltpu.VMEM((K,), jnp.int32),
            pltpu.VMEM((K,), jnp.int32),
            pltpu.VMEM((K, H), jnp.float32),
            pltpu.VMEM((K, H), jnp.float32),
            pltpu.VMEM((K, H), jnp.float32),
            pltpu.VMEM((K, H), jnp.float32),
            pltpu.VMEM_SHARED((NP, H), jnp.float32),
            pltpu.SemaphoreType.DMA,
            pltpu.SemaphoreType.DMA,
            pltpu.SemaphoreType.DMA,
            pltpu.SemaphoreType.DMA,
            pltpu.SemaphoreType.DMA,
            pltpu.SemaphoreType.DMA,
            pltpu.SemaphoreType.DMA,
            pltpu.SemaphoreType.DMA,
        ],
    )
    return f(x, e, src, dst)


BE = 2000


def _edge_lin1_body(ev_ref, w_ref, b_ref, e_ref):
    e_ref[...] = (jnp.dot(ev_ref[...], w_ref[...],
                          preferred_element_type=jnp.float32) + b_ref[...])


def _edge_lin2_body(ev_ref, w1_ref, w2_ref, b1_ref, b2_ref, e1_ref, e2_ref):
    ev = ev_ref[...]
    e1_ref[...] = jnp.dot(ev, w1_ref[...], preferred_element_type=jnp.float32) + b1_ref[...]
    e2_ref[...] = jnp.dot(ev, w2_ref[...], preferred_element_type=jnp.float32) + b2_ref[...]


_WSPEC = pl.BlockSpec((D_EDGE, H), lambda i: (0, 0))
_BSPEC = pl.BlockSpec((1, H), lambda i: (0, 0))
_EVSPEC = pl.BlockSpec((BE, D_EDGE), lambda i: (i, 0))
_ESPEC = pl.BlockSpec((BE, H), lambda i: (i, 0))


def _edge_linear1(ev, w, b):
    return pl.pallas_call(
        _edge_lin1_body,
        grid=(E // BE,),
        in_specs=[_EVSPEC, _WSPEC, _BSPEC],
        out_specs=_ESPEC,
        out_shape=jax.ShapeDtypeStruct((E, H), jnp.float32),
    )(ev, w, b)


def _edge_linear2(ev, w1, w2, b1, b2):
    return pl.pallas_call(
        _edge_lin2_body,
        grid=(E // BE,),
        in_specs=[_EVSPEC, _WSPEC, _WSPEC, _BSPEC, _BSPEC],
        out_specs=[_ESPEC, _ESPEC],
        out_shape=[jax.ShapeDtypeStruct((E, H), jnp.float32)] * 2,
    )(ev, w1, w2, b1, b2)


BN = 1000
GN = N // BN


def _mlp_body(t_ref, a0_ref, a1_ref, epsb_ref, w1_ref, b1_ref,
              bng_ref, bnb_ref, w2_ref, b2_ref, xres_ref, g_ref, b_ref,
              x_ref, t2_ref, h1_buf, stat):
    ph = pl.program_id(0)
    i = pl.program_id(1)

    @pl.when(ph == 0)
    def _():
        h = t_ref[...] * epsb_ref[...] + a0_ref[...] + a1_ref[...]
        h1 = jnp.dot(h, w1_ref[...], preferred_element_type=jnp.float32) + b1_ref[...]
        h1_buf[pl.ds(i * BN, BN), :] = h1
        ps = jnp.sum(h1, axis=0).reshape(1, H)
        pss = jnp.sum(h1 * h1, axis=0).reshape(1, H)

        @pl.when(i == 0)
        def _():
            stat[0:1, :] = ps
            stat[1:2, :] = pss

        @pl.when(i > 0)
        def _():
            stat[0:1, :] += ps
            stat[1:2, :] += pss

    @pl.when(ph == 1)
    def _():
        mu = stat[0:1, :] / N
        var = stat[1:2, :] / N - mu * mu
        inv = lax.rsqrt(var + 1e-5)
        h1 = h1_buf[pl.ds(i * BN, BN), :]
        h = (h1 - mu) * (inv * bng_ref[...]) + bnb_ref[...]
        h = jnp.maximum(h, 0.0)
        h2 = jnp.dot(h, w2_ref[...], preferred_element_type=jnp.float32) + b2_ref[...]
        x = xres_ref[...] + h2
        x_ref[...] = x
        mu_r = jnp.mean(x, axis=1, keepdims=True)
        xc = x - mu_r
        var_r = jnp.mean(xc * xc, axis=1, keepdims=True)
        t2 = g_ref[...] * xc * lax.rsqrt(var_r + 1e-5) + b_ref[...]
        t2_ref[...] = jnp.maximum(t2, 0.0)


def _mlp(t, a0, a1, epsb, w1, b1, bng, bnb, w2, b2, xres, g, b):
    nspec = pl.BlockSpec((BN, H), lambda ph, i: (i, 0))
    vspec = pl.BlockSpec((1, H), lambda ph, i: (0, 0))
    wspec = pl.BlockSpec((H, H), lambda ph, i: (0, 0))
    return pl.pallas_call(
        _mlp_body,
        grid=(2, GN),
        in_specs=[nspec, nspec, nspec, vspec, wspec, vspec,
                  vspec, vspec, wspec, vspec, nspec, vspec, vspec],
        out_specs=[nspec, nspec],
        out_shape=[jax.ShapeDtypeStruct((N, H), jnp.float32),
                   jax.ShapeDtypeStruct((N, H), jnp.float32)],
        scratch_shapes=[pltpu.VMEM((N, H), jnp.float32),
                        pltpu.VMEM((8, H), jnp.float32)],
    )(t, a0, a1, epsb, w1, b1, bng, bnb, w2, b2, xres, g, b)


def kernel(feature_vector, adj_index, edge_vector, params):
    p = params
    src = adj_index[0]
    dst = adj_index[1]

    e0 = _edge_linear1(edge_vector, p["We_0"].T, p["be_0"].reshape(1, H))
    norm_next = [("lng_1", "lnb_1"), ("lng_2", "lnb_2"), ("lng_0", "lnb_0")]

    t = feature_vector
    xres = jnp.zeros((N, H), jnp.float32)
    es = [e0, None, None]
    for k in range(3):
        parts = _sc_conv(t, es[k], src, dst)
        if k == 0:
            es[1], es[2] = _edge_linear2(
                edge_vector, p["We_1"].T, p["We_2"].T,
                p["be_1"].reshape(1, H), p["be_2"].reshape(1, H))
        epsb = jnp.broadcast_to(1.0 + p[f"eps_{k}"], (1, H)).astype(jnp.float32)
        gk, bk = norm_next[k]
        xres, t = _mlp(t, parts[:N], parts[NP:NP + N], epsb,
                       p[f"W1_{k}"].T, p[f"b1_{k}"].reshape(1, H),
                       p[f"bng_{k}"].reshape(1, H), p[f"bnb_{k}"].reshape(1, H),
                       p[f"W2_{k}"].T, p[f"b2_{k}"].reshape(1, H),
                       xres, p[gk].reshape(1, H), p[bk].reshape(1, H))
    return t


# speedup vs baseline: 1.0069x; 1.0069x over previous
import jax
import jax.numpy as jnp
from jax import lax
from jax.experimental import pallas as pl
from jax.experimental.pallas import tpu as pltpu
from jax.experimental.pallas import tpu_sc as plsc

N = 10000
E = 320000
H = 128
D_EDGE = 16
LANES = 16

NC = 2
NS = 16
NW = NC * NS
EPT = E // NW
K = 80
NCHUNK = EPT // K
NP = 10240
ZR = NP // NS


def _conv_body(x_hbm, e_hbm, src_hbm, dst_hbm, out_hbm,
               srcv0, srcv1, dstv0, dstv1,
               xrows0, xrows1, erows0, erows1, acc,
               sg0, sg1, se0, se1, ss0, ss1, si0, si1):
    c = lax.axis_index("c")
    s = lax.axis_index("s")
    wid = s * NC + c
    base = wid * EPT

    srcv = (srcv0, srcv1)
    dstv = (dstv0, dstv1)
    xrows = (xrows0, xrows1)
    erows = (erows0, erows1)
    sg = (sg0, sg1)
    se = (se0, se1)
    ss = (ss0, ss1)
    si = (si0, si1)

    def zrow(r, _):
        for j in range(H // LANES):
            xrows0[r, pl.ds(j * LANES, LANES)] = jnp.zeros((LANES,), jnp.float32)
        return 0
    lax.fori_loop(0, K, zrow, 0)
    for i in range(ZR // K):
        pltpu.async_copy(xrows0, acc.at[pl.ds(s * ZR + i * K, K)], sg0)
    for i in range(ZR // K):
        pltpu.make_async_copy(xrows0, acc.at[pl.ds(s * ZR, K)], sg0).wait()
    plsc.subcore_barrier()

    def issue_idx(slot, t):
        pltpu.async_copy(src_hbm.at[pl.ds(base + t * K, K)], srcv[slot], si[slot])
        pltpu.async_copy(dst_hbm.at[pl.ds(base + t * K, K)], dstv[slot], si[slot])

    def wait_idx(slot):
        pltpu.make_async_copy(src_hbm.at[pl.ds(base, K)], srcv[slot], si[slot]).wait()
        pltpu.make_async_copy(dst_hbm.at[pl.ds(base, K)], dstv[slot], si[slot]).wait()

    def issue_ge(slot, t):
        pltpu.async_copy(x_hbm.at[srcv[slot]], xrows[slot], sg[slot])
        pltpu.async_copy(e_hbm.at[pl.ds(base + t * K, K)], erows[slot], se[slot])

    def wait_ge(slot):
        pltpu.make_async_copy(x_hbm.at[srcv[slot]], xrows[slot], sg[slot]).wait()
        pltpu.make_async_copy(e_hbm.at[pl.ds(0, K)], erows[slot], se[slot]).wait()

    def compute(slot):
        xr = xrows[slot]
        er = erows[slot]

        def ew(q, _):
            for r in (2 * q, 2 * q + 1):
                for j in range(H // LANES):
                    sl = pl.ds(j * LANES, LANES)
                    xr[r, sl] = jnp.maximum(xr[r, sl] + er[r, sl], 0.0)
            return 0
        lax.fori_loop(0, K // 2, ew, 0)

    def scatter(slot):
        pltpu.async_copy(xrows[slot], acc.at[dstv[slot]], ss[slot], add=True)

    def wait_scatter(slot):
        pltpu.make_async_copy(xrows[slot], acc.at[dstv[slot]], ss[slot]).wait()

    issue_idx(0, 0)
    wait_idx(0)
    issue_idx(1, 1)
    issue_ge(0, 0)
    wait_ge(0)
    compute(0)
    scatter(0)
    wait_idx(1)
    issue_ge(1, 1)
    pltpu.make_async_copy(xrows[0], acc.at[dstv[0]], ss[0]).wait()
    issue_idx(0, 2)

    def pair(i, _):
        for b in range(2):
            t = 1 + 2 * i + b
            slot = (1 + b) % 2
            other = 1 - slot
            wait_ge(slot)
            compute(slot)
            scatter(slot)

            @pl.when(t < NCHUNK - 1)
            def _():
                wait_idx(other)
                issue_ge(other, t + 1)

            @pl.when(t < NCHUNK - 2)
            def _():
                wait_scatter(slot)
                issue_idx(slot, t + 2)
        return 0
    lax.fori_loop(0, (NCHUNK - 1) // 2, pair, 0)

    wait_scatter(0)
    wait_scatter(1)

    plsc.subcore_barrier()
    row0 = c * NP + s * ZR
    pltpu.sync_copy(acc.at[pl.ds(s * ZR, ZR)], out_hbm.at[pl.ds(row0, ZR)])


def _sc_conv(x, e, src, dst):
    f = pl.kernel(
        _conv_body,
        out_type=jax.ShapeDtypeStruct((2 * NP, H), jnp.float32),
        mesh=plsc.VectorSubcoreMesh(core_axis_name="c", subcore_axis_name="s"),
        scratch_types=[
            pltpu.VMEM((K,), jnp.int32),
            pltpu.VMEM((K,), jnp.int32),
            pltpu.VMEM((K,), jnp.int32),
            pltpu.VMEM((K,), jnp.int32),
            pltpu.VMEM((K, H), jnp.float32),
            pltpu.VMEM((K, H), jnp.float32),
            pltpu.VMEM((K, H), jnp.float32),
            pltpu.VMEM((K, H), jnp.float32),
            pltpu.VMEM_SHARED((NP, H), jnp.float32),
            pltpu.SemaphoreType.DMA,
            pltpu.SemaphoreType.DMA,
            pltpu.SemaphoreType.DMA,
            pltpu.SemaphoreType.DMA,
            pltpu.SemaphoreType.DMA,
            pltpu.SemaphoreType.DMA,
            pltpu.SemaphoreType.DMA,
            pltpu.SemaphoreType.DMA,
        ],
    )
    return f(x, e, src, dst)


BE = 2000


def _edge_lin1_body(ev_ref, w_ref, b_ref, e_ref):
    e_ref[...] = (jnp.dot(ev_ref[...], w_ref[...],
                          preferred_element_type=jnp.float32) + b_ref[...])


def _edge_lin2_body(ev_ref, w1_ref, w2_ref, b1_ref, b2_ref, e1_ref, e2_ref):
    ev = ev_ref[...]
    e1_ref[...] = jnp.dot(ev, w1_ref[...], preferred_element_type=jnp.float32) + b1_ref[...]
    e2_ref[...] = jnp.dot(ev, w2_ref[...], preferred_element_type=jnp.float32) + b2_ref[...]


_WSPEC = pl.BlockSpec((D_EDGE, H), lambda i: (0, 0))
_BSPEC = pl.BlockSpec((1, H), lambda i: (0, 0))
_EVSPEC = pl.BlockSpec((BE, D_EDGE), lambda i: (i, 0))
_ESPEC = pl.BlockSpec((BE, H), lambda i: (i, 0))


def _edge_linear1(ev, w, b):
    return pl.pallas_call(
        _edge_lin1_body,
        grid=(E // BE,),
        in_specs=[_EVSPEC, _WSPEC, _BSPEC],
        out_specs=_ESPEC,
        out_shape=jax.ShapeDtypeStruct((E, H), jnp.float32),
    )(ev, w, b)


def _edge_linear2(ev, w1, w2, b1, b2):
    return pl.pallas_call(
        _edge_lin2_body,
        grid=(E // BE,),
        in_specs=[_EVSPEC, _WSPEC, _WSPEC, _BSPEC, _BSPEC],
        out_specs=[_ESPEC, _ESPEC],
        out_shape=[jax.ShapeDtypeStruct((E, H), jnp.float32)] * 2,
    )(ev, w1, w2, b1, b2)


BN = 1000
GN = N // BN


def _mlpA_body(t_ref, a0_ref, a1_ref, epsb_ref, w1_ref, b1_ref,
               h1_ref, ps_ref, pss_ref):
    h = t_ref[...] * epsb_ref[...] + a0_ref[...] + a1_ref[...]
    h1 = jnp.dot(h, w1_ref[...], preferred_element_type=jnp.float32) + b1_ref[...]
    h1_ref[...] = h1
    ps_ref[...] = jnp.sum(h1, axis=0).reshape(1, 1, H)
    pss_ref[...] = jnp.sum(h1 * h1, axis=0).reshape(1, 1, H)


def _mlpA(t, a0, a1, epsb, w1, b1):
    nspec = pl.BlockSpec((BN, H), lambda i: (i, 0))
    return pl.pallas_call(
        _mlpA_body,
        grid=(GN,),
        in_specs=[nspec, nspec, nspec,
                  pl.BlockSpec((1, H), lambda i: (0, 0)),
                  pl.BlockSpec((H, H), lambda i: (0, 0)),
                  pl.BlockSpec((1, H), lambda i: (0, 0))],
        out_specs=[nspec,
                   pl.BlockSpec((1, 1, H), lambda i: (i, 0, 0)),
                   pl.BlockSpec((1, 1, H), lambda i: (i, 0, 0))],
        out_shape=[jax.ShapeDtypeStruct((N, H), jnp.float32),
                   jax.ShapeDtypeStruct((GN, 1, H), jnp.float32),
                   jax.ShapeDtypeStruct((GN, 1, H), jnp.float32)],
    )(t, a0, a1, epsb, w1, b1)


def _mlpB_body(h1_ref, ps_ref, pss_ref, bng_ref, bnb_ref, w2_ref, b2_ref,
               xres_ref, g_ref, b_ref, x_ref, t_ref):
    s1 = jnp.sum(ps_ref[...], axis=(0, 1))
    s2 = jnp.sum(pss_ref[...], axis=(0, 1))
    mu = s1 / N
    var = s2 / N - mu * mu
    inv = lax.rsqrt(var + 1e-5)
    h = (h1_ref[...] - mu) * (inv * bng_ref[...]) + bnb_ref[...]
    h = jnp.maximum(h, 0.0)
    h2 = jnp.dot(h, w2_ref[...], preferred_element_type=jnp.float32) + b2_ref[...]
    x = xres_ref[...] + h2
    x_ref[...] = x
    mu_r = jnp.mean(x, axis=1, keepdims=True)
    xc = x - mu_r
    var_r = jnp.mean(xc * xc, axis=1, keepdims=True)
    t = g_ref[...] * xc * lax.rsqrt(var_r + 1e-5) + b_ref[...]
    t_ref[...] = jnp.maximum(t, 0.0)


def _mlpB(h1, ps, pss, bng, bnb, w2, b2, xres, g, b):
    nspec = pl.BlockSpec((BN, H), lambda i: (i, 0))
    vspec = pl.BlockSpec((1, H), lambda i: (0, 0))
    return pl.pallas_call(
        _mlpB_body,
        grid=(GN,),
        in_specs=[nspec,
                  pl.BlockSpec((GN, 1, H), lambda i: (0, 0, 0)),
                  pl.BlockSpec((GN, 1, H), lambda i: (0, 0, 0)),
                  vspec, vspec,
                  pl.BlockSpec((H, H), lambda i: (0, 0)),
                  vspec, nspec, vspec, vspec],
        out_specs=[nspec, nspec],
        out_shape=[jax.ShapeDtypeStruct((N, H), jnp.float32),
                   jax.ShapeDtypeStruct((N, H), jnp.float32)],
    )(h1, ps, pss, bng, bnb, w2, b2, xres, g, b)


def kernel(feature_vector, adj_index, edge_vector, params):
    p = params
    src = adj_index[0]
    dst = adj_index[1]

    e0 = _edge_linear1(edge_vector, p["We_0"].T, p["be_0"].reshape(1, H))
    norm_next = [("lng_1", "lnb_1"), ("lng_2", "lnb_2"), ("lng_0", "lnb_0")]

    t = feature_vector
    xres = jnp.zeros((N, H), jnp.float32)
    es = [e0, None, None]
    for k in range(3):
        parts = _sc_conv(t, es[k], src, dst)
        if k == 0:
            es[1], es[2] = _edge_linear2(
                edge_vector, p["We_1"].T, p["We_2"].T,
                p["be_1"].reshape(1, H), p["be_2"].reshape(1, H))
        epsb = jnp.broadcast_to(1.0 + p[f"eps_{k}"], (1, H)).astype(jnp.float32)
        h1, ps, pss = _mlpA(t, parts[:N], parts[NP:NP + N], epsb,
                            p[f"W1_{k}"].T, p[f"b1_{k}"].reshape(1, H))
        gk, bk = norm_next[k]
        xres, t = _mlpB(h1, ps, pss,
                        p[f"bng_{k}"].reshape(1, H), p[f"bnb_{k}"].reshape(1, H),
                        p[f"W2_{k}"].T, p[f"b2_{k}"].reshape(1, H),
                        xres, p[gk].reshape(1, H), p[bk].reshape(1, H))
    return t
